# Initial kernel scaffold; baseline (speedup 1.0000x reference)
import jax
import jax.numpy as jnp
from jax.experimental import pallas as pl


def kernel(costs, valid, occ):
    raise NotImplementedError("write your pallas kernel here")


# speedup vs baseline: 2817.6591x; 2817.6591x over previous
import functools

import jax
import jax.numpy as jnp
from jax import lax
from jax.experimental import pallas as pl
from jax.experimental.pallas import tpu as pltpu
from jax.experimental.pallas import tpu_sc as plsc

P, Q, S = 64, 64, 4096
L = 16
NC, NS = 2, 16
NW = NC * NS
PPW = P // NW
QB = 8
NCHUNK = Q // QB
SCH = S // L

_mesh = plsc.VectorSubcoreMesh(core_axis_name="c", subcore_axis_name="s")


@functools.partial(
    pl.kernel,
    mesh=_mesh,
    out_type=jax.ShapeDtypeStruct((P, Q), jnp.float32),
    compiler_params=pltpu.CompilerParams(needs_layout_passes=False),
    scratch_types=[
        pltpu.VMEM((S,), jnp.float32),
        pltpu.VMEM((P,), jnp.float32),
        pltpu.VMEM((Q,), jnp.float32),
        pltpu.VMEM((QB, S), jnp.float32),
        pltpu.VMEM((QB, S), jnp.float32),
        pltpu.SemaphoreType.DMA,
        pltpu.SemaphoreType.DMA,
    ],
)
def _sc_logits(costs_hbm, valid_hbm, occ_hbm, out_hbm,
               costs_v, valid_v, out_v, buf0, buf1, sem0, sem1):
    wid = lax.axis_index("s") * NC + lax.axis_index("c")
    pltpu.sync_copy(valid_hbm, valid_v)
    bufs = (buf0, buf1)
    sems = (sem0, sem1)
    lane_iota = lax.iota(jnp.int32, L)
    for t in range(PPW):
        p = wid * PPW + t
        pltpu.sync_copy(costs_hbm.at[p], costs_v)

        def _clean(si, carry):
            base = pl.multiple_of(si * L, L)
            c = costs_v[pl.ds(base, L)]
            costs_v[pl.ds(base, L)] = jnp.where(c != c, jnp.float32(1.0), c)
            return carry

        lax.fori_loop(0, SCH, _clean, 0)

        pbase = pl.multiple_of((p // L) * L, L)
        vvec = valid_v[pl.ds(pbase, L)]
        vfv = jnp.sum(jnp.where(lane_iota == (p - pbase), vvec,
                                jnp.float32(0.0)))

        handles = [None, None]
        handles[0] = pltpu.async_copy(
            occ_hbm.at[p, pl.ds(0, QB), :], buf0, sem0)
        res = jnp.zeros((L,), jnp.float32)
        for qc in range(NCHUNK):
            if qc + 1 < NCHUNK:
                nb = (qc + 1) % 2
                handles[nb] = pltpu.async_copy(
                    occ_hbm.at[p, pl.ds((qc + 1) * QB, QB), :], bufs[nb], sems[nb])
            handles[qc % 2].wait()
            buf = bufs[qc % 2]

            def _acc(si, accs):
                base = pl.multiple_of(si * L, L)
                c = costs_v[pl.ds(base, L)]
                return tuple(accs[j] + buf[j, pl.ds(base, L)] * c
                             for j in range(QB))

            accs = lax.fori_loop(
                0, SCH, _acc,
                tuple(jnp.zeros((L,), jnp.float32) for _ in range(QB)))
            for j in range(QB):
                res = jnp.where(lane_iota == ((qc % 2) * QB + j),
                                jnp.sum(accs[j]), res)
            if qc % 2 == 1:
                out_v[pl.ds((qc // 2) * L, L)] = res * vfv
                res = jnp.zeros((L,), jnp.float32)
        pltpu.sync_copy(out_v, out_hbm.at[p])


def kernel(costs, valid, occ):
    logits = _sc_logits(costs, valid.astype(jnp.float32), occ)
    return (logits, valid)
